# baseline (speedup 1.0000x reference)
import dataclasses
import functools

import jax
import jax.numpy as jnp
from jax import lax
from jax.experimental import pallas as pl
from jax.experimental.pallas import tpu as pltpu
from jax.experimental.pallas import tpu_sc as plsc

_B, _E, _HW = 8, 32, 224 * 224
_K = 16
_DELTA_VAR, _DELTA_DIST = 0.5, 1.5
_ALPHA, _BETA, _GAMMA = 1.0, 1.0, 0.001
_PB = 7168
_NB = _HW // _PB

_NW = 32
_WPB = _NW // _B
_CPW = _HW // _WPB
_CH = 896
_NCH = _CPW // _CH
_NG = _CH // 16


def _sc_segment_kernel(e_hbm, lab_hbm, out_hbm, eb0, eb1, lb0, lb1,
                       acc, cacc, outv, sem0, sem1):
    wid = lax.axis_index("c") * 16 + lax.axis_index("s")
    b = wid // _WPB
    q = wid % _WPB

    iota = lax.iota(jnp.int32, 16)
    lane512 = iota * 512
    zeros16 = jnp.zeros((16,), jnp.float32)
    ones16 = jnp.ones((16,), jnp.float32)

    for i in range(512):
        acc[pl.ds(i * 16, 16)] = zeros16
    for i in range(16):
        cacc[pl.ds(i * 16, 16)] = zeros16

    ebufs, lbufs, sems = (eb0, eb1), (lb0, lb1), (sem0, sem1)

    def copies(k):
        i = k % 2
        off = q * _CPW + k * _CH
        ce = pltpu.make_async_copy(
            e_hbm.at[b, :, pl.ds(off, _CH)], ebufs[i], sems[i])
        cl = pltpu.make_async_copy(
            lab_hbm.at[b, pl.ds(off, _CH)], lbufs[i], sems[i])
        return ce, cl

    def start(k):
        ce, cl = copies(k)
        ce.start()
        cl.start()

    def process(k):
        i = k % 2
        ev, lv_ref = ebufs[i], lbufs[i]

        @pl.loop(0, _NG)
        def _(g):
            p0 = pl.multiple_of(g * 16, 16)
            lv = lv_ref[pl.ds(p0, 16)]
            base = lane512 + lv * 32
            cidx = iota * 16 + lv
            plsc.addupdate_scatter(cacc, [cidx], ones16)
            for c in range(_E):
                x = ev[c, pl.ds(p0, 16)]
                plsc.addupdate_scatter(acc, [base + c], x)

    start(0)
    for k in range(_NCH):
        if k + 1 < _NCH:
            start(k + 1)
        ce, cl = copies(k)
        ce.wait()
        cl.wait()
        process(k)

    for i in range(32):
        s = acc[pl.ds(i * 16, 16)]
        for j in range(1, 16):
            s = s + acc[pl.ds(j * 512 + i * 16, 16)]
        outv[pl.ds(i * 16, 16)] = s
    cnt = cacc[pl.ds(0, 16)]
    for j in range(1, 16):
        cnt = cnt + cacc[pl.ds(j * 16, 16)]
    for c in range(_E):
        plsc.store_scatter(outv, [512 + iota * 32 + c], cnt)

    pltpu.sync_copy(outv, out_hbm.at[wid])


@jax.jit
def _sc_segment(e3, lab2):
    mesh = plsc.VectorSubcoreMesh(core_axis_name="c", subcore_axis_name="s")
    cp = pltpu.CompilerParams()
    if "needs_layout_passes" in pltpu.CompilerParams.__dataclass_fields__:
        cp = dataclasses.replace(cp, needs_layout_passes=False)
    k = functools.partial(
        pl.kernel,
        out_type=jax.ShapeDtypeStruct((_NW, 1024), jnp.float32),
        mesh=mesh,
        compiler_params=cp,
        scratch_types=[
            pltpu.VMEM((_E, _CH), jnp.float32),
            pltpu.VMEM((_E, _CH), jnp.float32),
            pltpu.VMEM((_CH,), jnp.int32),
            pltpu.VMEM((_CH,), jnp.int32),
            pltpu.VMEM((8192,), jnp.float32),
            pltpu.VMEM((256,), jnp.float32),
            pltpu.VMEM((1024,), jnp.float32),
            pltpu.SemaphoreType.DMA,
            pltpu.SemaphoreType.DMA,
        ],
    )(_sc_segment_kernel)
    return k(e3, lab2)


def _tc_loss_kernel(psc_ref, lab_ref, e_ref, out_ref, acc_ref):
    j = pl.program_id(1)

    e = e_ref[0]
    lab = lab_ref[0]
    kio = lax.broadcasted_iota(jnp.int32, (_K, _PB), 0)
    oh = (kio == lab).astype(jnp.float32)

    @pl.when(j == 0)
    def _init():
        acc_ref[0] = 0.0

    ps = psc_ref[0]
    sums = ps[0, 0] + ps[1, 0] + ps[2, 0] + ps[3, 0]
    cntm = ps[0, 1] + ps[1, 1] + ps[2, 1] + ps[3, 1]
    means = sums / jnp.maximum(cntm, 1.0)
    mm2 = jnp.sum(means * means, axis=1, keepdims=True)

    dot2 = lax.dot_general(2.0 * means, e, (((1,), (0,)), ((), ())),
                           precision=lax.Precision.DEFAULT)
    sel = jnp.sum(oh * (mm2 - dot2), axis=0, keepdims=True)
    ee = jnp.sum(e * e, axis=0, keepdims=True)
    d2 = ee + sel
    dist = jnp.sqrt(jnp.maximum(d2, 0.0))
    valid = (lab > 0).astype(jnp.float32)
    hinge = jnp.maximum(dist - _DELTA_VAR, 0.0) * valid
    acc_ref[0] += jnp.sum(hinge)

    @pl.when(j == _NB - 1)
    def _finalize():
        cnt = cntm[:, 0:1]
        ids = lax.broadcasted_iota(jnp.int32, (_K, 1), 0)
        presf = jnp.logical_and(cnt > 0.0, ids > 0).astype(jnp.float32)
        n_inst = jnp.sum(presf)
        var_loss = acc_ref[0] / jnp.maximum(n_inst, 1.0)
        g = lax.dot_general(means, means, (((1,), (1,)), ((), ())),
                            precision=lax.Precision.HIGHEST)
        p2 = lax.dot_general(presf, presf, (((1,), (1,)), ((), ())),
                             precision=lax.Precision.HIGHEST)
        ir = lax.broadcasted_iota(jnp.int32, (_K, _K), 0)
        ic = lax.broadcasted_iota(jnp.int32, (_K, _K), 1)
        eye = (ir == ic).astype(jnp.float32)
        ge = g * eye
        mm2c = jnp.sum(ge, axis=1, keepdims=True)
        mm2r = jnp.sum(ge, axis=0, keepdims=True)
        pd2 = mm2c + mm2r - 2.0 * g
        pd = jnp.sqrt(jnp.maximum(pd2, 0.0))
        hingep = jnp.maximum(2.0 * _DELTA_DIST - pd, 0.0)
        tri = (ir < ic).astype(jnp.float32)
        pairsum = jnp.sum(hingep * p2 * tri)
        n_pairs = n_inst * (n_inst - 1.0) * 0.5
        dist_loss = jnp.where(n_inst > 1.0,
                              pairsum / jnp.maximum(n_pairs, 1.0), 0.0)
        mnorm = jnp.sqrt(jnp.maximum(mm2, 0.0))
        reg_loss = jnp.sum(presf * mnorm) / jnp.maximum(n_inst, 1.0)
        total = (_ALPHA * var_loss + _BETA * dist_loss + _GAMMA * reg_loss)
        out_ref[...] = jnp.broadcast_to(total, (1, 1, 1))


def kernel(embeddings, instance_labels):
    e3 = embeddings.reshape(_B, _E, _HW)
    lab2 = instance_labels.reshape(_B, _HW)
    psc = _sc_segment(e3, lab2)
    psc5 = psc.reshape(_B, _WPB, 2, _K, _E)
    lab3 = instance_labels.reshape(_B * _NB, 1, _PB)
    per_batch = pl.pallas_call(
        _tc_loss_kernel,
        grid=(_B, _NB),
        in_specs=[
            pl.BlockSpec((1, _WPB, 2, _K, _E), lambda b, j: (b, 0, 0, 0, 0)),
            pl.BlockSpec((1, 1, _PB), lambda b, j: (b * _NB + j, 0, 0)),
            pl.BlockSpec((1, _E, _PB), lambda b, j: (b, 0, j)),
        ],
        out_specs=pl.BlockSpec((1, 1, 1), lambda b, j: (b, 0, 0)),
        out_shape=jax.ShapeDtypeStruct((_B, 1, 1), jnp.float32),
        scratch_shapes=[
            pltpu.SMEM((1,), jnp.float32),
        ],
    )(psc5, lab3, e3)
    return jnp.sum(per_batch) / _B


# speedup vs baseline: 1.1504x; 1.1504x over previous
import dataclasses
import functools

import jax
import jax.numpy as jnp
from jax import lax
from jax.experimental import pallas as pl
from jax.experimental.pallas import tpu as pltpu
from jax.experimental.pallas import tpu_sc as plsc

_B, _E, _HW = 8, 32, 224 * 224
_K = 16
_DELTA_VAR, _DELTA_DIST = 0.5, 1.5
_ALPHA, _BETA, _GAMMA = 1.0, 1.0, 0.001
_PB = 7168
_NB = _HW // _PB

_NW = 32
_WPB = _NW // _B
_CPW = _HW // _WPB
_CH = 896
_NCH = _CPW // _CH
_NG = _CH // 16


def _sc_segment_kernel(e_hbm, lab_hbm, out_hbm, eb0, eb1, lb0, lb1,
                       acc, cacc, outv, sem0, sem1):
    wid = lax.axis_index("c") * 16 + lax.axis_index("s")
    b = wid // _WPB
    q = wid % _WPB

    iota = lax.iota(jnp.int32, 16)
    lane512 = iota * 512
    zeros16 = jnp.zeros((16,), jnp.float32)
    ones16 = jnp.ones((16,), jnp.float32)

    for i in range(512):
        acc[pl.ds(i * 16, 16)] = zeros16
    for i in range(16):
        cacc[pl.ds(i * 16, 16)] = zeros16

    ebufs, lbufs, sems = (eb0, eb1), (lb0, lb1), (sem0, sem1)

    def copies(k):
        i = k % 2
        off = q * _CPW + k * _CH
        ce = pltpu.make_async_copy(
            e_hbm.at[b, :, pl.ds(off, _CH)], ebufs[i], sems[i])
        cl = pltpu.make_async_copy(
            lab_hbm.at[b, pl.ds(off, _CH)], lbufs[i], sems[i])
        return ce, cl

    def start(k):
        ce, cl = copies(k)
        ce.start()
        cl.start()

    def process(k):
        i = k % 2
        ev, lv_ref = ebufs[i], lbufs[i]

        @plsc.parallel_loop(0, _NG, unroll=2)
        def _(g):
            p0 = pl.multiple_of(g * 16, 16)
            lv = lv_ref[pl.ds(p0, 16)]
            base = lane512 + lv * 32
            cidx = iota * 16 + lv
            plsc.addupdate_scatter(cacc, [cidx], ones16)
            for h in range(0, _E, 16):
                vals = [ev[c, pl.ds(p0, 16)] for c in range(h, h + 16)]
                for t, c in enumerate(range(h, h + 16)):
                    plsc.addupdate_scatter(acc, [base + c], vals[t])

    start(0)
    for k in range(_NCH):
        if k + 1 < _NCH:
            start(k + 1)
        ce, cl = copies(k)
        ce.wait()
        cl.wait()
        process(k)

    for i in range(32):
        s = acc[pl.ds(i * 16, 16)]
        for j in range(1, 16):
            s = s + acc[pl.ds(j * 512 + i * 16, 16)]
        outv[pl.ds(i * 16, 16)] = s
    cnt = cacc[pl.ds(0, 16)]
    for j in range(1, 16):
        cnt = cnt + cacc[pl.ds(j * 16, 16)]
    for c in range(_E):
        plsc.store_scatter(outv, [512 + iota * 32 + c], cnt)

    pltpu.sync_copy(outv, out_hbm.at[wid])


@jax.jit
def _sc_segment(e3, lab2):
    mesh = plsc.VectorSubcoreMesh(core_axis_name="c", subcore_axis_name="s")
    cp = pltpu.CompilerParams()
    if "needs_layout_passes" in pltpu.CompilerParams.__dataclass_fields__:
        cp = dataclasses.replace(cp, needs_layout_passes=False)
    k = functools.partial(
        pl.kernel,
        out_type=jax.ShapeDtypeStruct((_NW, 1024), jnp.float32),
        mesh=mesh,
        compiler_params=cp,
        scratch_types=[
            pltpu.VMEM((_E, _CH), jnp.float32),
            pltpu.VMEM((_E, _CH), jnp.float32),
            pltpu.VMEM((_CH,), jnp.int32),
            pltpu.VMEM((_CH,), jnp.int32),
            pltpu.VMEM((8192,), jnp.float32),
            pltpu.VMEM((256,), jnp.float32),
            pltpu.VMEM((1024,), jnp.float32),
            pltpu.SemaphoreType.DMA,
            pltpu.SemaphoreType.DMA,
        ],
    )(_sc_segment_kernel)
    return k(e3, lab2)


def _tc_loss_kernel(psc_ref, lab_ref, e_ref, out_ref, acc_ref):
    j = pl.program_id(1)

    e = e_ref[0]
    lab = lab_ref[0]
    kio = lax.broadcasted_iota(jnp.int32, (_K, _PB), 0)
    oh = (kio == lab).astype(jnp.float32)

    @pl.when(j == 0)
    def _init():
        acc_ref[0] = 0.0

    ps = psc_ref[0]
    sums = ps[0, 0] + ps[1, 0] + ps[2, 0] + ps[3, 0]
    cntm = ps[0, 1] + ps[1, 1] + ps[2, 1] + ps[3, 1]
    means = sums / jnp.maximum(cntm, 1.0)
    mm2 = jnp.sum(means * means, axis=1, keepdims=True)

    dot2 = lax.dot_general(2.0 * means, e, (((1,), (0,)), ((), ())),
                           precision=lax.Precision.DEFAULT)
    sel = jnp.sum(oh * (mm2 - dot2), axis=0, keepdims=True)
    ee = jnp.sum(e * e, axis=0, keepdims=True)
    d2 = ee + sel
    dist = jnp.sqrt(jnp.maximum(d2, 0.0))
    valid = (lab > 0).astype(jnp.float32)
    hinge = jnp.maximum(dist - _DELTA_VAR, 0.0) * valid
    acc_ref[0] += jnp.sum(hinge)

    @pl.when(j == _NB - 1)
    def _finalize():
        cnt = cntm[:, 0:1]
        ids = lax.broadcasted_iota(jnp.int32, (_K, 1), 0)
        presf = jnp.logical_and(cnt > 0.0, ids > 0).astype(jnp.float32)
        n_inst = jnp.sum(presf)
        var_loss = acc_ref[0] / jnp.maximum(n_inst, 1.0)
        g = lax.dot_general(means, means, (((1,), (1,)), ((), ())),
                            precision=lax.Precision.HIGHEST)
        p2 = lax.dot_general(presf, presf, (((1,), (1,)), ((), ())),
                             precision=lax.Precision.HIGHEST)
        ir = lax.broadcasted_iota(jnp.int32, (_K, _K), 0)
        ic = lax.broadcasted_iota(jnp.int32, (_K, _K), 1)
        eye = (ir == ic).astype(jnp.float32)
        ge = g * eye
        mm2c = jnp.sum(ge, axis=1, keepdims=True)
        mm2r = jnp.sum(ge, axis=0, keepdims=True)
        pd2 = mm2c + mm2r - 2.0 * g
        pd = jnp.sqrt(jnp.maximum(pd2, 0.0))
        hingep = jnp.maximum(2.0 * _DELTA_DIST - pd, 0.0)
        tri = (ir < ic).astype(jnp.float32)
        pairsum = jnp.sum(hingep * p2 * tri)
        n_pairs = n_inst * (n_inst - 1.0) * 0.5
        dist_loss = jnp.where(n_inst > 1.0,
                              pairsum / jnp.maximum(n_pairs, 1.0), 0.0)
        mnorm = jnp.sqrt(jnp.maximum(mm2, 0.0))
        reg_loss = jnp.sum(presf * mnorm) / jnp.maximum(n_inst, 1.0)
        total = (_ALPHA * var_loss + _BETA * dist_loss + _GAMMA * reg_loss)
        out_ref[...] = jnp.broadcast_to(total, (1, 1, 1))


def kernel(embeddings, instance_labels):
    e3 = embeddings.reshape(_B, _E, _HW)
    lab2 = instance_labels.reshape(_B, _HW)
    psc = _sc_segment(e3, lab2)
    psc5 = psc.reshape(_B, _WPB, 2, _K, _E)
    lab3 = instance_labels.reshape(_B * _NB, 1, _PB)
    per_batch = pl.pallas_call(
        _tc_loss_kernel,
        grid=(_B, _NB),
        in_specs=[
            pl.BlockSpec((1, _WPB, 2, _K, _E), lambda b, j: (b, 0, 0, 0, 0)),
            pl.BlockSpec((1, 1, _PB), lambda b, j: (b * _NB + j, 0, 0)),
            pl.BlockSpec((1, _E, _PB), lambda b, j: (b, 0, j)),
        ],
        out_specs=pl.BlockSpec((1, 1, 1), lambda b, j: (b, 0, 0)),
        out_shape=jax.ShapeDtypeStruct((_B, 1, 1), jnp.float32),
        scratch_shapes=[
            pltpu.SMEM((1,), jnp.float32),
        ],
    )(psc5, lab3, e3)
    return jnp.sum(per_batch) / _B


# speedup vs baseline: 2.5439x; 2.2114x over previous
import dataclasses
import functools

import jax
import jax.numpy as jnp
from jax import lax
from jax.experimental import pallas as pl
from jax.experimental.pallas import tpu as pltpu
from jax.experimental.pallas import tpu_sc as plsc

_B, _E, _HW = 8, 32, 224 * 224
_K = 16
_DELTA_VAR, _DELTA_DIST = 0.5, 1.5
_ALPHA, _BETA, _GAMMA = 1.0, 1.0, 0.001
_PB = 7168
_NB = _HW // _PB

_NW = 32
_WPB = _NW // _B
_CPW = _HW // _WPB
_CH = 896
_NCH = _CPW // _CH
_NG = _CH // 16


def _sc_segment_kernel(e_hbm, lab_hbm, out_hbm, eb0, eb1, lb0, lb1,
                       acc, cacc, outv, sem0, sem1):
    wid = lax.axis_index("c") * 16 + lax.axis_index("s")
    b = wid // _WPB
    q = wid % _WPB

    iota = lax.iota(jnp.int32, 16)
    iota16 = iota * 16
    zeros16 = jnp.zeros((16,), jnp.float32)
    ones16 = jnp.ones((16,), jnp.float32)

    for i in range(512):
        acc[pl.ds(i * 16, 16)] = zeros16
    for i in range(16):
        cacc[pl.ds(i * 16, 16)] = zeros16

    ebufs, lbufs, sems = (eb0, eb1), (lb0, lb1), (sem0, sem1)

    def copies(k):
        i = k % 2
        off = q * _CPW + k * _CH
        ce = pltpu.make_async_copy(
            e_hbm.at[b, :, pl.ds(off, _CH)], ebufs[i], sems[i])
        cl = pltpu.make_async_copy(
            lab_hbm.at[b, pl.ds(off, _CH)], lbufs[i], sems[i])
        return ce, cl

    def start(k):
        ce, cl = copies(k)
        ce.start()
        cl.start()

    def process(k):
        i = k % 2
        ev, lv_ref = ebufs[i], lbufs[i]

        @plsc.parallel_loop(0, _NG, unroll=2)
        def _(g):
            p0 = pl.multiple_of(g * 16, 16)
            lv = lv_ref[pl.ds(p0, 16)]
            base = lv * (_E * 16) + iota
            cidx = lv * 16 + iota
            plsc.addupdate_scatter(cacc, [cidx], ones16)
            for h in range(0, _E, 16):
                vals = [ev[c, pl.ds(p0, 16)] for c in range(h, h + 16)]
                for t, c in enumerate(range(h, h + 16)):
                    plsc.addupdate_scatter(acc, [base + c * 16], vals[t])

    start(0)
    for k in range(_NCH):
        if k + 1 < _NCH:
            start(k + 1)
        ce, cl = copies(k)
        ce.wait()
        cl.wait()
        process(k)

    for p in range(32):
        s = zeros16
        for j in range(16):
            s = s + plsc.load_gather(acc, [iota16 + (p * 256 + j)])
        outv[pl.ds(p * 16, 16)] = s
    cnt = zeros16
    for j in range(16):
        cnt = cnt + plsc.load_gather(cacc, [iota16 + j])
    for c in range(_E):
        plsc.store_scatter(outv, [512 + iota * 32 + c], cnt)

    pltpu.sync_copy(outv, out_hbm.at[wid])


@jax.jit
def _sc_segment(e3, lab2):
    mesh = plsc.VectorSubcoreMesh(core_axis_name="c", subcore_axis_name="s")
    cp = pltpu.CompilerParams()
    if "needs_layout_passes" in pltpu.CompilerParams.__dataclass_fields__:
        cp = dataclasses.replace(cp, needs_layout_passes=False)
    k = functools.partial(
        pl.kernel,
        out_type=jax.ShapeDtypeStruct((_NW, 1024), jnp.float32),
        mesh=mesh,
        compiler_params=cp,
        scratch_types=[
            pltpu.VMEM((_E, _CH), jnp.float32),
            pltpu.VMEM((_E, _CH), jnp.float32),
            pltpu.VMEM((_CH,), jnp.int32),
            pltpu.VMEM((_CH,), jnp.int32),
            pltpu.VMEM((8192,), jnp.float32),
            pltpu.VMEM((256,), jnp.float32),
            pltpu.VMEM((1024,), jnp.float32),
            pltpu.SemaphoreType.DMA,
            pltpu.SemaphoreType.DMA,
        ],
    )(_sc_segment_kernel)
    return k(e3, lab2)


def _tc_loss_kernel(psc_ref, lab_ref, e_ref, out_ref, acc_ref):
    j = pl.program_id(1)

    e = e_ref[0]
    lab = lab_ref[0]
    kio = lax.broadcasted_iota(jnp.int32, (_K, _PB), 0)
    oh = (kio == lab).astype(jnp.float32)

    @pl.when(j == 0)
    def _init():
        acc_ref[0] = 0.0

    ps = psc_ref[0]
    sums = ps[0, 0] + ps[1, 0] + ps[2, 0] + ps[3, 0]
    cntm = ps[0, 1] + ps[1, 1] + ps[2, 1] + ps[3, 1]
    means = sums / jnp.maximum(cntm, 1.0)
    mm2 = jnp.sum(means * means, axis=1, keepdims=True)

    dot2 = lax.dot_general(2.0 * means, e, (((1,), (0,)), ((), ())),
                           precision=lax.Precision.DEFAULT)
    sel = jnp.sum(oh * (mm2 - dot2), axis=0, keepdims=True)
    ee = jnp.sum(e * e, axis=0, keepdims=True)
    d2 = ee + sel
    dist = jnp.sqrt(jnp.maximum(d2, 0.0))
    valid = (lab > 0).astype(jnp.float32)
    hinge = jnp.maximum(dist - _DELTA_VAR, 0.0) * valid
    acc_ref[0] += jnp.sum(hinge)

    @pl.when(j == _NB - 1)
    def _finalize():
        cnt = cntm[:, 0:1]
        ids = lax.broadcasted_iota(jnp.int32, (_K, 1), 0)
        presf = jnp.logical_and(cnt > 0.0, ids > 0).astype(jnp.float32)
        n_inst = jnp.sum(presf)
        var_loss = acc_ref[0] / jnp.maximum(n_inst, 1.0)
        g = lax.dot_general(means, means, (((1,), (1,)), ((), ())),
                            precision=lax.Precision.HIGHEST)
        p2 = lax.dot_general(presf, presf, (((1,), (1,)), ((), ())),
                             precision=lax.Precision.HIGHEST)
        ir = lax.broadcasted_iota(jnp.int32, (_K, _K), 0)
        ic = lax.broadcasted_iota(jnp.int32, (_K, _K), 1)
        eye = (ir == ic).astype(jnp.float32)
        ge = g * eye
        mm2c = jnp.sum(ge, axis=1, keepdims=True)
        mm2r = jnp.sum(ge, axis=0, keepdims=True)
        pd2 = mm2c + mm2r - 2.0 * g
        pd = jnp.sqrt(jnp.maximum(pd2, 0.0))
        hingep = jnp.maximum(2.0 * _DELTA_DIST - pd, 0.0)
        tri = (ir < ic).astype(jnp.float32)
        pairsum = jnp.sum(hingep * p2 * tri)
        n_pairs = n_inst * (n_inst - 1.0) * 0.5
        dist_loss = jnp.where(n_inst > 1.0,
                              pairsum / jnp.maximum(n_pairs, 1.0), 0.0)
        mnorm = jnp.sqrt(jnp.maximum(mm2, 0.0))
        reg_loss = jnp.sum(presf * mnorm) / jnp.maximum(n_inst, 1.0)
        total = (_ALPHA * var_loss + _BETA * dist_loss + _GAMMA * reg_loss)
        out_ref[...] = jnp.broadcast_to(total, (1, 1, 1))


def kernel(embeddings, instance_labels):
    e3 = embeddings.reshape(_B, _E, _HW)
    lab2 = instance_labels.reshape(_B, _HW)
    psc = _sc_segment(e3, lab2)
    psc5 = psc.reshape(_B, _WPB, 2, _K, _E)
    lab3 = instance_labels.reshape(_B * _NB, 1, _PB)
    per_batch = pl.pallas_call(
        _tc_loss_kernel,
        grid=(_B, _NB),
        in_specs=[
            pl.BlockSpec((1, _WPB, 2, _K, _E), lambda b, j: (b, 0, 0, 0, 0)),
            pl.BlockSpec((1, 1, _PB), lambda b, j: (b * _NB + j, 0, 0)),
            pl.BlockSpec((1, _E, _PB), lambda b, j: (b, 0, j)),
        ],
        out_specs=pl.BlockSpec((1, 1, 1), lambda b, j: (b, 0, 0)),
        out_shape=jax.ShapeDtypeStruct((_B, 1, 1), jnp.float32),
        scratch_shapes=[
            pltpu.SMEM((1,), jnp.float32),
        ],
    )(psc5, lab3, e3)
    return jnp.sum(per_batch) / _B


# speedup vs baseline: 2.7584x; 1.0843x over previous
import dataclasses
import functools

import jax
import jax.numpy as jnp
from jax import lax
from jax.experimental import pallas as pl
from jax.experimental.pallas import tpu as pltpu
from jax.experimental.pallas import tpu_sc as plsc

_B, _E, _HW = 8, 32, 224 * 224
_K = 16
_DELTA_VAR, _DELTA_DIST = 0.5, 1.5
_ALPHA, _BETA, _GAMMA = 1.0, 1.0, 0.001
_PB = 12544
_NB = _HW // _PB

_NW = 32
_WPB = _NW // _B
_CPW = _HW // _WPB
_CH = 896
_NCH = _CPW // _CH
_NG = _CH // 16


def _sc_segment_kernel(e_hbm, lab_hbm, out_hbm, eb0, eb1, lb0, lb1,
                       acc, cacc, outv, sem0, sem1):
    wid = lax.axis_index("c") * 16 + lax.axis_index("s")
    b = wid // _WPB
    q = wid % _WPB

    iota = lax.iota(jnp.int32, 16)
    iota16 = iota * 16
    zeros16 = jnp.zeros((16,), jnp.float32)
    ones16 = jnp.ones((16,), jnp.float32)

    for i in range(512):
        acc[pl.ds(i * 16, 16)] = zeros16
    for i in range(16):
        cacc[pl.ds(i * 16, 16)] = zeros16

    ebufs, lbufs, sems = (eb0, eb1), (lb0, lb1), (sem0, sem1)

    def copies(k):
        i = k % 2
        off = q * _CPW + k * _CH
        ce = pltpu.make_async_copy(
            e_hbm.at[b, :, pl.ds(off, _CH)], ebufs[i], sems[i])
        cl = pltpu.make_async_copy(
            lab_hbm.at[b, pl.ds(off, _CH)], lbufs[i], sems[i])
        return ce, cl

    def start(k):
        ce, cl = copies(k)
        ce.start()
        cl.start()

    def process(k):
        i = k % 2
        ev, lv_ref = ebufs[i], lbufs[i]

        @plsc.parallel_loop(0, _NG, unroll=2)
        def _(g):
            p0 = pl.multiple_of(g * 16, 16)
            lv = lv_ref[pl.ds(p0, 16)]
            base = lv * (_E * 16) + iota
            cidx = lv * 16 + iota
            plsc.addupdate_scatter(cacc, [cidx], ones16)
            for h in range(0, _E, 16):
                vals = [ev[c, pl.ds(p0, 16)] for c in range(h, h + 16)]
                for t, c in enumerate(range(h, h + 16)):
                    plsc.addupdate_scatter(acc, [base + c * 16], vals[t])

    start(0)
    for k in range(_NCH):
        if k + 1 < _NCH:
            start(k + 1)
        ce, cl = copies(k)
        ce.wait()
        cl.wait()
        process(k)

    for p in range(32):
        s = zeros16
        for j in range(16):
            s = s + plsc.load_gather(acc, [iota16 + (p * 256 + j)])
        outv[pl.ds(p * 16, 16)] = s
    cnt = zeros16
    for j in range(16):
        cnt = cnt + plsc.load_gather(cacc, [iota16 + j])
    for c in range(_E):
        plsc.store_scatter(outv, [512 + iota * 32 + c], cnt)

    pltpu.sync_copy(outv, out_hbm.at[wid])


@jax.jit
def _sc_segment(e3, lab2):
    mesh = plsc.VectorSubcoreMesh(core_axis_name="c", subcore_axis_name="s")
    cp = pltpu.CompilerParams()
    if "needs_layout_passes" in pltpu.CompilerParams.__dataclass_fields__:
        cp = dataclasses.replace(cp, needs_layout_passes=False)
    k = functools.partial(
        pl.kernel,
        out_type=jax.ShapeDtypeStruct((_NW, 1024), jnp.float32),
        mesh=mesh,
        compiler_params=cp,
        scratch_types=[
            pltpu.VMEM((_E, _CH), jnp.float32),
            pltpu.VMEM((_E, _CH), jnp.float32),
            pltpu.VMEM((_CH,), jnp.int32),
            pltpu.VMEM((_CH,), jnp.int32),
            pltpu.VMEM((8192,), jnp.float32),
            pltpu.VMEM((256,), jnp.float32),
            pltpu.VMEM((1024,), jnp.float32),
            pltpu.SemaphoreType.DMA,
            pltpu.SemaphoreType.DMA,
        ],
    )(_sc_segment_kernel)
    return k(e3, lab2)


def _tc_loss_kernel(psc_ref, lab_ref, e_ref, out_ref, acc_ref):
    j = pl.program_id(1)

    e = e_ref[0]
    lab = lab_ref[0]
    kio = lax.broadcasted_iota(jnp.int32, (_K, _PB), 0)
    oh = (kio == lab).astype(jnp.float32)

    @pl.when(j == 0)
    def _init():
        acc_ref[0] = 0.0

    ps = psc_ref[0]
    sums = ps[0, 0] + ps[1, 0] + ps[2, 0] + ps[3, 0]
    cntm = ps[0, 1] + ps[1, 1] + ps[2, 1] + ps[3, 1]
    means = sums / jnp.maximum(cntm, 1.0)
    mm2 = jnp.sum(means * means, axis=1, keepdims=True)

    dot2 = lax.dot_general(2.0 * means, e, (((1,), (0,)), ((), ())),
                           precision=lax.Precision.DEFAULT)
    sel = jnp.sum(oh * (mm2 - dot2), axis=0, keepdims=True)
    ee = jnp.sum(e * e, axis=0, keepdims=True)
    d2 = ee + sel
    dist = jnp.sqrt(jnp.maximum(d2, 0.0))
    valid = (lab > 0).astype(jnp.float32)
    hinge = jnp.maximum(dist - _DELTA_VAR, 0.0) * valid
    acc_ref[0] += jnp.sum(hinge)

    @pl.when(j == _NB - 1)
    def _finalize():
        cnt = cntm[:, 0:1]
        ids = lax.broadcasted_iota(jnp.int32, (_K, 1), 0)
        presf = jnp.logical_and(cnt > 0.0, ids > 0).astype(jnp.float32)
        n_inst = jnp.sum(presf)
        var_loss = acc_ref[0] / jnp.maximum(n_inst, 1.0)
        g = lax.dot_general(means, means, (((1,), (1,)), ((), ())),
                            precision=lax.Precision.HIGHEST)
        p2 = lax.dot_general(presf, presf, (((1,), (1,)), ((), ())),
                             precision=lax.Precision.HIGHEST)
        ir = lax.broadcasted_iota(jnp.int32, (_K, _K), 0)
        ic = lax.broadcasted_iota(jnp.int32, (_K, _K), 1)
        eye = (ir == ic).astype(jnp.float32)
        ge = g * eye
        mm2c = jnp.sum(ge, axis=1, keepdims=True)
        mm2r = jnp.sum(ge, axis=0, keepdims=True)
        pd2 = mm2c + mm2r - 2.0 * g
        pd = jnp.sqrt(jnp.maximum(pd2, 0.0))
        hingep = jnp.maximum(2.0 * _DELTA_DIST - pd, 0.0)
        tri = (ir < ic).astype(jnp.float32)
        pairsum = jnp.sum(hingep * p2 * tri)
        n_pairs = n_inst * (n_inst - 1.0) * 0.5
        dist_loss = jnp.where(n_inst > 1.0,
                              pairsum / jnp.maximum(n_pairs, 1.0), 0.0)
        mnorm = jnp.sqrt(jnp.maximum(mm2, 0.0))
        reg_loss = jnp.sum(presf * mnorm) / jnp.maximum(n_inst, 1.0)
        total = (_ALPHA * var_loss + _BETA * dist_loss + _GAMMA * reg_loss)
        out_ref[...] = jnp.broadcast_to(total, (1, 1, 1))


def kernel(embeddings, instance_labels):
    e3 = embeddings.reshape(_B, _E, _HW)
    lab2 = instance_labels.reshape(_B, _HW)
    psc = _sc_segment(e3, lab2)
    psc5 = psc.reshape(_B, _WPB, 2, _K, _E)
    lab3 = instance_labels.reshape(_B * _NB, 1, _PB)
    per_batch = pl.pallas_call(
        _tc_loss_kernel,
        grid=(_B, _NB),
        in_specs=[
            pl.BlockSpec((1, _WPB, 2, _K, _E), lambda b, j: (b, 0, 0, 0, 0)),
            pl.BlockSpec((1, 1, _PB), lambda b, j: (b * _NB + j, 0, 0)),
            pl.BlockSpec((1, _E, _PB), lambda b, j: (b, 0, j)),
        ],
        out_specs=pl.BlockSpec((1, 1, 1), lambda b, j: (b, 0, 0)),
        out_shape=jax.ShapeDtypeStruct((_B, 1, 1), jnp.float32),
        scratch_shapes=[
            pltpu.SMEM((1,), jnp.float32),
        ],
    )(psc5, lab3, e3)
    return jnp.sum(per_batch) / _B


# speedup vs baseline: 3.0186x; 1.0943x over previous
import dataclasses
import functools

import jax
import jax.numpy as jnp
from jax import lax
from jax.experimental import pallas as pl
from jax.experimental.pallas import tpu as pltpu
from jax.experimental.pallas import tpu_sc as plsc

_B, _E, _HW = 8, 32, 224 * 224
_K = 16
_DELTA_VAR, _DELTA_DIST = 0.5, 1.5
_ALPHA, _BETA, _GAMMA = 1.0, 1.0, 0.001
_PB = 12544
_NB = _HW // _PB

_NW = 32
_WPB = _NW // _B
_CPW = _HW // _WPB
_W = 224
_RPW = _CPW // _W
_CHR = 4
_CH = _CHR * _W
_NCH = _RPW // _CHR
_NG = _CH // 16


def _sc_segment_kernel(e_hbm, lab_hbm, out_hbm, eb0, eb1,
                       lb0, lb1, acc, cacc, outv, sem0, sem1):
    wid = lax.axis_index("c") * 16 + lax.axis_index("s")
    b = wid // _WPB
    q = wid % _WPB

    iota = lax.iota(jnp.int32, 16)
    iota16 = iota * 16
    zeros16 = jnp.zeros((16,), jnp.float32)
    ones16 = jnp.ones((16,), jnp.float32)

    for i in range(512):
        acc[pl.ds(i * 16, 16)] = zeros16
    for i in range(16):
        cacc[pl.ds(i * 16, 16)] = zeros16

    ebufs, lbufs = (eb0, eb1), (lb0, lb1)
    sems = (sem0, sem1)

    def copies(k):
        i = k % 2
        r0 = q * _RPW + k * _CHR
        return [
            pltpu.make_async_copy(
                e_hbm.at[b, :, pl.ds(r0, _CHR), :], ebufs[i], sems[i]),
            pltpu.make_async_copy(
                lab_hbm.at[b, pl.ds(r0, _CHR), :], lbufs[i], sems[i]),
        ]

    def start(k):
        for c in copies(k):
            c.start()

    def process(k):
        i = k % 2
        ev, lv_ref = ebufs[i], lbufs[i]

        @plsc.parallel_loop(0, _NG, unroll=2)
        def _(g):
            rr = g // (_W // 16)
            c0 = pl.multiple_of((g % (_W // 16)) * 16, 16)
            lv = lv_ref[rr, pl.ds(c0, 16)]
            base = lv * (_E * 16) + iota
            cidx = lv * 16 + iota
            plsc.addupdate_scatter(cacc, [cidx], ones16)
            for h in range(0, _E, 16):
                vals = [ev[c, rr, pl.ds(c0, 16)] for c in range(h, h + 16)]
                for t, c in enumerate(range(h, h + 16)):
                    plsc.addupdate_scatter(acc, [base + c * 16], vals[t])

    start(0)
    for k in range(_NCH):
        if k + 1 < _NCH:
            start(k + 1)
        for c in copies(k):
            c.wait()
        process(k)

    for p in range(32):
        s = zeros16
        for j in range(16):
            s = s + plsc.load_gather(acc, [iota16 + (p * 256 + j)])
        outv[pl.ds(p * 16, 16)] = s
    cnt = zeros16
    for j in range(16):
        cnt = cnt + plsc.load_gather(cacc, [iota16 + j])
    for c in range(_E):
        plsc.store_scatter(outv, [512 + iota * 32 + c], cnt)

    pltpu.sync_copy(outv, out_hbm.at[wid])


@jax.jit
def _sc_segment(e4, lab3d):
    mesh = plsc.VectorSubcoreMesh(core_axis_name="c", subcore_axis_name="s")
    cp = pltpu.CompilerParams()
    if "needs_layout_passes" in pltpu.CompilerParams.__dataclass_fields__:
        cp = dataclasses.replace(cp, needs_layout_passes=False)
    k = functools.partial(
        pl.kernel,
        out_type=jax.ShapeDtypeStruct((_NW, 1024), jnp.float32),
        mesh=mesh,
        compiler_params=cp,
        scratch_types=[
            pltpu.VMEM((_E, _CHR, _W), jnp.float32),
            pltpu.VMEM((_E, _CHR, _W), jnp.float32),
            pltpu.VMEM((_CHR, _W), jnp.int32),
            pltpu.VMEM((_CHR, _W), jnp.int32),
            pltpu.VMEM((8192,), jnp.float32),
            pltpu.VMEM((256,), jnp.float32),
            pltpu.VMEM((1024,), jnp.float32),
            pltpu.SemaphoreType.DMA,
            pltpu.SemaphoreType.DMA,
        ],
    )(_sc_segment_kernel)
    return k(e4, lab3d)


def _tc_loss_kernel(psc_ref, lab_ref, e_ref, out_ref, acc_ref):
    j = pl.program_id(1)

    e = e_ref[0]
    lab = lab_ref[0]
    kio = lax.broadcasted_iota(jnp.int32, (_K, _PB), 0)
    oh = (kio == lab).astype(jnp.float32)

    @pl.when(j == 0)
    def _init():
        acc_ref[0] = 0.0

    ps = psc_ref[0]
    sums = ps[0, 0] + ps[1, 0] + ps[2, 0] + ps[3, 0]
    cntm = ps[0, 1] + ps[1, 1] + ps[2, 1] + ps[3, 1]
    means = sums / jnp.maximum(cntm, 1.0)
    mm2 = jnp.sum(means * means, axis=1, keepdims=True)

    dot2 = lax.dot_general(2.0 * means, e, (((1,), (0,)), ((), ())),
                           precision=lax.Precision.DEFAULT)
    sel = jnp.sum(oh * (mm2 - dot2), axis=0, keepdims=True)
    ee = jnp.sum(e * e, axis=0, keepdims=True)
    d2 = ee + sel
    dist = jnp.sqrt(jnp.maximum(d2, 0.0))
    valid = (lab > 0).astype(jnp.float32)
    hinge = jnp.maximum(dist - _DELTA_VAR, 0.0) * valid
    acc_ref[0] += jnp.sum(hinge)

    @pl.when(j == _NB - 1)
    def _finalize():
        cnt = cntm[:, 0:1]
        ids = lax.broadcasted_iota(jnp.int32, (_K, 1), 0)
        presf = jnp.logical_and(cnt > 0.0, ids > 0).astype(jnp.float32)
        n_inst = jnp.sum(presf)
        var_loss = acc_ref[0] / jnp.maximum(n_inst, 1.0)
        g = lax.dot_general(means, means, (((1,), (1,)), ((), ())),
                            precision=lax.Precision.HIGHEST)
        p2 = lax.dot_general(presf, presf, (((1,), (1,)), ((), ())),
                             precision=lax.Precision.HIGHEST)
        ir = lax.broadcasted_iota(jnp.int32, (_K, _K), 0)
        ic = lax.broadcasted_iota(jnp.int32, (_K, _K), 1)
        eye = (ir == ic).astype(jnp.float32)
        ge = g * eye
        mm2c = jnp.sum(ge, axis=1, keepdims=True)
        mm2r = jnp.sum(ge, axis=0, keepdims=True)
        pd2 = mm2c + mm2r - 2.0 * g
        pd = jnp.sqrt(jnp.maximum(pd2, 0.0))
        hingep = jnp.maximum(2.0 * _DELTA_DIST - pd, 0.0)
        tri = (ir < ic).astype(jnp.float32)
        pairsum = jnp.sum(hingep * p2 * tri)
        n_pairs = n_inst * (n_inst - 1.0) * 0.5
        dist_loss = jnp.where(n_inst > 1.0,
                              pairsum / jnp.maximum(n_pairs, 1.0), 0.0)
        mnorm = jnp.sqrt(jnp.maximum(mm2, 0.0))
        reg_loss = jnp.sum(presf * mnorm) / jnp.maximum(n_inst, 1.0)
        total = (_ALPHA * var_loss + _BETA * dist_loss + _GAMMA * reg_loss)
        out_ref[...] = jnp.broadcast_to(total, (1, 1, 1))


def kernel(embeddings, instance_labels):
    psc = _sc_segment(embeddings, instance_labels)
    e3 = embeddings.reshape(_B, _E, _HW)
    psc5 = psc.reshape(_B, _WPB, 2, _K, _E)
    lab3 = instance_labels.reshape(_B * _NB, 1, _PB)
    per_batch = pl.pallas_call(
        _tc_loss_kernel,
        grid=(_B, _NB),
        in_specs=[
            pl.BlockSpec((1, _WPB, 2, _K, _E), lambda b, j: (b, 0, 0, 0, 0)),
            pl.BlockSpec((1, 1, _PB), lambda b, j: (b * _NB + j, 0, 0)),
            pl.BlockSpec((1, _E, _PB), lambda b, j: (b, 0, j)),
        ],
        out_specs=pl.BlockSpec((1, 1, 1), lambda b, j: (b, 0, 0)),
        out_shape=jax.ShapeDtypeStruct((_B, 1, 1), jnp.float32),
        scratch_shapes=[
            pltpu.SMEM((1,), jnp.float32),
        ],
    )(psc5, lab3, e3)
    return jnp.sum(per_batch) / _B
